# Initial kernel scaffold; baseline (speedup 1.0000x reference)
import jax
import jax.numpy as jnp
from jax.experimental import pallas as pl


def kernel(x, batch, W, b):
    raise NotImplementedError("write your pallas kernel here")


# speedup vs baseline: 1.9968x; 1.9968x over previous
import functools

import jax
import jax.numpy as jnp
from jax import lax
from jax.experimental import pallas as pl
from jax.experimental.pallas import tpu as pltpu
from jax.experimental.pallas import tpu_sc as plsc

N = 50000
D = 512
G = 128

NC = 2
NS = 16

DH = D // NC
T = 64
RPW = 3136
NTILES = RPW // T
NP = NS * RPW
GP = 144

HG = DH // 16


def _sc_pool(xp, bp):
  mesh = plsc.VectorSubcoreMesh(core_axis_name="c", subcore_axis_name="s")

  @functools.partial(
      pl.kernel,
      out_type=(
          jax.ShapeDtypeStruct((GP * D,), jnp.float32),
          jax.ShapeDtypeStruct((GP * D,), jnp.float32),
          jax.ShapeDtypeStruct((GP * 16,), jnp.float32),
      ),
      mesh=mesh,
      scratch_types=dict(
          buf=pltpu.VMEM((T, DH), jnp.float32),
          cur_idx=pltpu.VMEM((T,), jnp.int32),
          slab_s=pltpu.VMEM((GP * DH,), jnp.float32),
          slab_m=pltpu.VMEM((GP * DH,), jnp.float32),
          slab_c=pltpu.VMEM((GP * 16,), jnp.float32),
          tmp=pltpu.VMEM((DH,), jnp.float32),
          acc=pltpu.VMEM((DH,), jnp.float32),
          tmp_c=pltpu.VMEM((16,), jnp.float32),
          acc_c=pltpu.VMEM((16,), jnp.float32),
          stage_s=pltpu.VMEM_SHARED((NS * 16 * DH,), jnp.float32),
          stage_m=pltpu.VMEM_SHARED((NS * 16 * DH,), jnp.float32),
          stage_c=pltpu.VMEM_SHARED((NS * 16 * 16,), jnp.float32),
      ),
  )
  def k(x_hbm, b_hbm, out_sum, out_max, out_cnt,
        buf, cur_idx, slab_s, slab_m, slab_c, tmp, acc, tmp_c, acc_c,
        stage_s, stage_m, stage_c):
    c = lax.axis_index("c")
    s = lax.axis_index("s")
    col0 = c * DH

    zero = jnp.zeros((16,), jnp.float32)
    neg = jnp.full((16,), -jnp.inf, jnp.float32)

    def fill(i, _):
      slab_s[pl.ds(i * 16, 16)] = zero
      slab_m[pl.ds(i * 16, 16)] = neg
      return 0

    lax.fori_loop(0, GP * HG, fill, 0)

    def fillc(i, _):
      slab_c[pl.ds(i * 16, 16)] = zero
      return 0

    lax.fori_loop(0, GP, fillc, 0)

    one = jnp.full((16,), 1.0, jnp.float32)

    def tile_body(t, _):
      rowbase = s * RPW + t * T
      pltpu.sync_copy(x_hbm.at[pl.ds(rowbase, T), pl.ds(col0, DH)], buf)
      pltpu.sync_copy(b_hbm.at[pl.ds(rowbase, T)], cur_idx)

      for sg in range(T // 16):
        bvec = cur_idx[pl.ds(sg * 16, 16)]
        bases = bvec * DH
        cbases = bvec * 16

        for l in range(16):
          bb = bases[l]
          cb = cbases[l]
          r = sg * 16 + l
          cnt = slab_c[pl.ds(cb, 16)]
          slab_c[pl.ds(cb, 16)] = cnt + one

          def grp_body(g, _, r=r, bb=bb):
            col = g * 16
            v = buf[r, pl.ds(col, 16)]
            m = slab_m[pl.ds(bb + col, 16)]
            slab_m[pl.ds(bb + col, 16)] = jnp.maximum(m, v)
            sm = slab_s[pl.ds(bb + col, 16)]
            slab_s[pl.ds(bb + col, 16)] = sm + v
            return 0

          lax.fori_loop(0, HG, grp_body, 0)
      return 0

    lax.fori_loop(0, NTILES, tile_body, 0)

    for ch in range(GP // 16):
      r = ch * 16 + s
      pltpu.sync_copy(slab_s.at[pl.ds(ch * 16 * DH, 16 * DH)],
                      stage_s.at[pl.ds(s * 16 * DH, 16 * DH)])
      pltpu.sync_copy(slab_m.at[pl.ds(ch * 16 * DH, 16 * DH)],
                      stage_m.at[pl.ds(s * 16 * DH, 16 * DH)])
      pltpu.sync_copy(slab_c.at[pl.ds(ch * 16 * 16, 16 * 16)],
                      stage_c.at[pl.ds(s * 16 * 16, 16 * 16)])
      plsc.subcore_barrier()

      pltpu.sync_copy(stage_s.at[pl.ds(s * DH, DH)], acc)

      def sum_body(j, _, s=s):
        pltpu.sync_copy(stage_s.at[pl.ds(j * 16 * DH + s * DH, DH)], tmp)
        for g in range(HG):
          acc[pl.ds(g * 16, 16)] = acc[pl.ds(g * 16, 16)] + tmp[pl.ds(g * 16, 16)]
        return 0

      lax.fori_loop(1, NS, sum_body, 0)
      pltpu.sync_copy(acc, out_sum.at[pl.ds(r * D + col0, DH)])

      pltpu.sync_copy(stage_m.at[pl.ds(s * DH, DH)], acc)

      def max_body(j, _, s=s):
        pltpu.sync_copy(stage_m.at[pl.ds(j * 16 * DH + s * DH, DH)], tmp)
        for g in range(HG):
          acc[pl.ds(g * 16, 16)] = jnp.maximum(acc[pl.ds(g * 16, 16)],
                                               tmp[pl.ds(g * 16, 16)])
        return 0

      lax.fori_loop(1, NS, max_body, 0)
      pltpu.sync_copy(acc, out_max.at[pl.ds(r * D + col0, DH)])

      @pl.when(c == 0)
      def _():
        pltpu.sync_copy(stage_c.at[pl.ds(s * 16, 16)], acc_c)

        def cnt_body(j, _, s=s):
          pltpu.sync_copy(stage_c.at[pl.ds(j * 16 * 16 + s * 16, 16)], tmp_c)
          acc_c[pl.ds(0, 16)] = acc_c[pl.ds(0, 16)] + tmp_c[pl.ds(0, 16)]
          return 0

        lax.fori_loop(1, NS, cnt_body, 0)
        pltpu.sync_copy(acc_c, out_cnt.at[pl.ds(r * 16, 16)])

      plsc.subcore_barrier()

  return k(xp, bp)


def _tc_combine(sums, cnts, maxs, W, b):
  def body(sum_ref, cnt_ref, max_ref, w_ref, b_ref, out_ref):
    cnt = cnt_ref[:G, 0:1]
    mean = sum_ref[:G, :] / jnp.maximum(cnt, 1.0)
    mx = max_ref[:G, :]
    wm = w_ref[:, 0:D] + w_ref[:, 2 * D:3 * D] + w_ref[:, 4 * D:5 * D]
    wx = w_ref[:, D:2 * D] + w_ref[:, 3 * D:4 * D] + w_ref[:, 5 * D:6 * D]
    dn = (((1,), (1,)), ((), ()))
    out = lax.dot_general(mean, wm, dn, preferred_element_type=jnp.float32)
    out += lax.dot_general(mx, wx, dn, preferred_element_type=jnp.float32)
    out_ref[...] = out + b_ref[0, :]

  return pl.pallas_call(
      body,
      out_shape=jax.ShapeDtypeStruct((G, D), jnp.float32),
  )(sums, cnts, maxs, W, b.reshape(1, D))


def kernel(x, batch, W, b):
  xp = jnp.concatenate(
      [x, jnp.zeros((NP - N, D), jnp.float32)], axis=0)
  bp = jnp.concatenate(
      [batch, jnp.full((NP - N,), G, jnp.int32)], axis=0)

  sums, maxs, cnts = _sc_pool(xp, bp)
  return _tc_combine(sums.reshape(GP, D), cnts.reshape(GP, 16),
                     maxs.reshape(GP, D), W, b)


# speedup vs baseline: 2.8839x; 1.4443x over previous
import functools

import jax
import jax.numpy as jnp
from jax import lax
from jax.experimental import pallas as pl
from jax.experimental.pallas import tpu as pltpu
from jax.experimental.pallas import tpu_sc as plsc

N = 50000
D = 512
G = 128

NC = 2
NS = 16

DH = D // NC
T = 32
RPW = 3136
NTILES = RPW // T
NPAIRS = NTILES // 2
NP = NS * RPW
GP = 144

HG = DH // 16


def _sc_pool(xp, bp):
  mesh = plsc.VectorSubcoreMesh(core_axis_name="c", subcore_axis_name="s")

  @functools.partial(
      pl.kernel,
      out_type=(
          jax.ShapeDtypeStruct((GP * D,), jnp.float32),
          jax.ShapeDtypeStruct((GP * D,), jnp.float32),
          jax.ShapeDtypeStruct((GP * 16,), jnp.float32),
      ),
      mesh=mesh,
      scratch_types=dict(
          buf0=pltpu.VMEM((T, DH), jnp.float32),
          buf1=pltpu.VMEM((T, DH), jnp.float32),
          idx0=pltpu.VMEM((T,), jnp.int32),
          idx1=pltpu.VMEM((T,), jnp.int32),
          slab_s=pltpu.VMEM((GP * DH,), jnp.float32),
          slab_m=pltpu.VMEM((GP * DH,), jnp.float32),
          slab_c=pltpu.VMEM((GP * 16,), jnp.float32),
          tmps=pltpu.VMEM((NS * DH,), jnp.float32),
          acc=pltpu.VMEM((DH,), jnp.float32),
          tmp_c=pltpu.VMEM((16,), jnp.float32),
          acc_c=pltpu.VMEM((16,), jnp.float32),
          stage_s=pltpu.VMEM_SHARED((NS * 16 * DH,), jnp.float32),
          stage_m=pltpu.VMEM_SHARED((NS * 16 * DH,), jnp.float32),
          stage_c=pltpu.VMEM_SHARED((NS * 16 * 16,), jnp.float32),
          sem0=pltpu.SemaphoreType.DMA,
          sem1=pltpu.SemaphoreType.DMA,
          sem2=pltpu.SemaphoreType.DMA,
      ),
  )
  def k(x_hbm, b_hbm, out_sum, out_max, out_cnt,
        buf0, buf1, idx0, idx1, slab_s, slab_m, slab_c, tmps, acc,
        tmp_c, acc_c, stage_s, stage_m, stage_c, sem0, sem1, sem2):
    c = lax.axis_index("c")
    s = lax.axis_index("s")
    col0 = c * DH

    zero = jnp.zeros((16,), jnp.float32)
    neg = jnp.full((16,), -jnp.inf, jnp.float32)

    def fill(i, _):
      slab_s[pl.ds(i * 16, 16)] = zero
      slab_m[pl.ds(i * 16, 16)] = neg
      return 0

    lax.fori_loop(0, GP * HG, fill, 0)

    def fillc(i, _):
      slab_c[pl.ds(i * 16, 16)] = zero
      return 0

    lax.fori_loop(0, GP, fillc, 0)

    one = jnp.full((16,), 1.0, jnp.float32)

    def start_tile(t, buf, idx, sem):
      rowbase = s * RPW + t * T
      pltpu.async_copy(x_hbm.at[pl.ds(rowbase, T), pl.ds(col0, DH)], buf, sem)
      pltpu.async_copy(b_hbm.at[pl.ds(rowbase, T)], idx, sem)

    def wait_tile(t, buf, idx, sem):
      rowbase = s * RPW + t * T
      pltpu.make_async_copy(
          x_hbm.at[pl.ds(rowbase, T), pl.ds(col0, DH)], buf, sem).wait()
      pltpu.make_async_copy(b_hbm.at[pl.ds(rowbase, T)], idx, sem).wait()

    def process(buf, cur_idx):
      for sg in range(T // 16):
        bvec = cur_idx[pl.ds(sg * 16, 16)]
        bases = bvec * DH
        cbases = bvec * 16

        for l in range(16):
          bb = bases[l]
          cb = cbases[l]
          r = sg * 16 + l
          cnt = slab_c[pl.ds(cb, 16)]
          slab_c[pl.ds(cb, 16)] = cnt + one

          def grp_body(q, _, r=r, bb=bb):
            for u in range(4):
              col = q * 64 + u * 16
              v = buf[r, pl.ds(col, 16)]
              m = slab_m[pl.ds(bb + col, 16)]
              slab_m[pl.ds(bb + col, 16)] = jnp.maximum(m, v)
              sm = slab_s[pl.ds(bb + col, 16)]
              slab_s[pl.ds(bb + col, 16)] = sm + v
            return 0

          lax.fori_loop(0, HG // 4, grp_body, 0)

    start_tile(0, buf0, idx0, sem0)

    def pair_body(i, _):
      t0 = 2 * i
      start_tile(t0 + 1, buf1, idx1, sem1)
      wait_tile(t0, buf0, idx0, sem0)
      process(buf0, idx0)

      @pl.when(i + 1 < NPAIRS)
      def _():
        start_tile(t0 + 2, buf0, idx0, sem0)

      wait_tile(t0 + 1, buf1, idx1, sem1)
      process(buf1, idx1)
      return 0

    lax.fori_loop(0, NPAIRS, pair_body, 0)

    def chunk_body(ch, _):
      r = ch * 16 + s
      pltpu.sync_copy(slab_s.at[pl.ds(ch * 16 * DH, 16 * DH)],
                      stage_s.at[pl.ds(s * 16 * DH, 16 * DH)])
      pltpu.sync_copy(slab_m.at[pl.ds(ch * 16 * DH, 16 * DH)],
                      stage_m.at[pl.ds(s * 16 * DH, 16 * DH)])
      pltpu.sync_copy(slab_c.at[pl.ds(ch * 16 * 16, 16 * 16)],
                      stage_c.at[pl.ds(s * 16 * 16, 16 * 16)])
      plsc.subcore_barrier()

      hs = [pltpu.async_copy(stage_s.at[pl.ds(j * 16 * DH + s * DH, DH)],
                             tmps.at[pl.ds(j * DH, DH)], sem2)
            for j in range(NS)]
      for h in hs:
        h.wait()

      def red_sum(g, _):
        a = tmps[pl.ds(g * 16, 16)]
        for j in range(1, NS):
          a = a + tmps[pl.ds(j * DH + g * 16, 16)]
        acc[pl.ds(g * 16, 16)] = a
        return 0

      lax.fori_loop(0, HG, red_sum, 0)
      pltpu.sync_copy(acc, out_sum.at[pl.ds(r * D + col0, DH)])

      hs = [pltpu.async_copy(stage_m.at[pl.ds(j * 16 * DH + s * DH, DH)],
                             tmps.at[pl.ds(j * DH, DH)], sem2)
            for j in range(NS)]
      for h in hs:
        h.wait()

      def red_max(g, _):
        a = tmps[pl.ds(g * 16, 16)]
        for j in range(1, NS):
          a = jnp.maximum(a, tmps[pl.ds(j * DH + g * 16, 16)])
        acc[pl.ds(g * 16, 16)] = a
        return 0

      lax.fori_loop(0, HG, red_max, 0)
      pltpu.sync_copy(acc, out_max.at[pl.ds(r * D + col0, DH)])

      @pl.when(c == 0)
      def _():
        pltpu.sync_copy(stage_c.at[pl.ds(s * 16, 16)], acc_c)

        def cnt_body(j, _):
          pltpu.sync_copy(stage_c.at[pl.ds(j * 16 * 16 + s * 16, 16)], tmp_c)
          acc_c[pl.ds(0, 16)] = acc_c[pl.ds(0, 16)] + tmp_c[pl.ds(0, 16)]
          return 0

        lax.fori_loop(1, NS, cnt_body, 0)
        pltpu.sync_copy(acc_c, out_cnt.at[pl.ds(r * 16, 16)])

      plsc.subcore_barrier()
      return 0

    lax.fori_loop(0, GP // 16, chunk_body, 0)

  return k(xp, bp)


def _tc_combine(sums, cnts, maxs, W, b):
  def body(sum_ref, cnt_ref, max_ref, w_ref, b_ref, out_ref):
    cnt = cnt_ref[:G, 0:1]
    mean = sum_ref[:G, :] / jnp.maximum(cnt, 1.0)
    mx = max_ref[:G, :]
    wm = w_ref[:, 0:D] + w_ref[:, 2 * D:3 * D] + w_ref[:, 4 * D:5 * D]
    wx = w_ref[:, D:2 * D] + w_ref[:, 3 * D:4 * D] + w_ref[:, 5 * D:6 * D]
    dn = (((1,), (1,)), ((), ()))
    out = lax.dot_general(mean, wm, dn, preferred_element_type=jnp.float32)
    out += lax.dot_general(mx, wx, dn, preferred_element_type=jnp.float32)
    out_ref[...] = out + b_ref[0, :]

  return pl.pallas_call(
      body,
      out_shape=jax.ShapeDtypeStruct((G, D), jnp.float32),
  )(sums, cnts, maxs, W, b.reshape(1, D))


def kernel(x, batch, W, b):
  xp = jnp.concatenate(
      [x, jnp.zeros((NP - N, D), jnp.float32)], axis=0)
  bp = jnp.concatenate(
      [batch, jnp.full((NP - N,), G, jnp.int32)], axis=0)

  sums, maxs, cnts = _sc_pool(xp, bp)
  return _tc_combine(sums.reshape(GP, D), cnts.reshape(GP, 16),
                     maxs.reshape(GP, D), W, b)


# speedup vs baseline: 8.0815x; 2.8023x over previous
import functools

import jax
import jax.numpy as jnp
from jax import lax
from jax.experimental import pallas as pl
from jax.experimental.pallas import tpu as pltpu
from jax.experimental.pallas import tpu_sc as plsc

N = 50000
D = 512
G = 128

NC = 2
NS = 16

DH = D // NC
T = 32
RPW = 3136
NTILES = RPW // T
NPAIRS = NTILES // 2
LAST_PAIRS = 46
TAIL0 = 49984
GP = 128

HG = DH // 16


def _sc_pool(xp, bp):
  mesh = plsc.VectorSubcoreMesh(core_axis_name="c", subcore_axis_name="s")

  @functools.partial(
      pl.kernel,
      out_type=(
          jax.ShapeDtypeStruct((NC * NS * GP * DH,), jnp.float32),
          jax.ShapeDtypeStruct((NC * NS * GP * DH,), jnp.float32),
          jax.ShapeDtypeStruct((NS * GP * 16,), jnp.float32),
      ),
      mesh=mesh,
      scratch_types=dict(
          buf0=pltpu.VMEM((T, DH), jnp.float32),
          buf1=pltpu.VMEM((T, DH), jnp.float32),
          idx0=pltpu.VMEM((T,), jnp.int32),
          idx1=pltpu.VMEM((T,), jnp.int32),
          slab_s=pltpu.VMEM((GP * DH,), jnp.float32),
          slab_m=pltpu.VMEM((GP * DH,), jnp.float32),
          slab_c=pltpu.VMEM((GP * 16,), jnp.float32),
          sem0=pltpu.SemaphoreType.DMA,
          sem1=pltpu.SemaphoreType.DMA,
      ),
  )
  def k(x_hbm, b_hbm, out_sum, out_max, out_cnt,
        buf0, buf1, idx0, idx1, slab_s, slab_m, slab_c, sem0, sem1):
    c = lax.axis_index("c")
    s = lax.axis_index("s")
    col0 = c * DH

    zero = jnp.zeros((16,), jnp.float32)
    neg = jnp.full((16,), -jnp.inf, jnp.float32)

    def fill(i, _):
      slab_s[pl.ds(i * 16, 16)] = zero
      slab_m[pl.ds(i * 16, 16)] = neg
      return 0

    lax.fori_loop(0, GP * HG, fill, 0)

    def fillc(i, _):
      slab_c[pl.ds(i * 16, 16)] = zero
      return 0

    lax.fori_loop(0, GP, fillc, 0)

    one = jnp.full((16,), 1.0, jnp.float32)

    def start_tile(t, buf, idx, sem):
      rowbase = s * RPW + t * T
      pltpu.async_copy(x_hbm.at[pl.ds(rowbase, T), pl.ds(col0, DH)], buf, sem)
      pltpu.async_copy(b_hbm.at[pl.ds(rowbase, T)], idx, sem)

    def wait_tile(t, buf, idx, sem):
      rowbase = s * RPW + t * T
      pltpu.make_async_copy(
          x_hbm.at[pl.ds(rowbase, T), pl.ds(col0, DH)], buf, sem).wait()
      pltpu.make_async_copy(b_hbm.at[pl.ds(rowbase, T)], idx, sem).wait()

    sixteen = jnp.full((16,), 16.0, jnp.float32)

    def do_sg(buf, cur_idx, sg):
      if True:
        bvec = cur_idx[pl.ds(sg * 16, 16)]
        bases = bvec * DH
        cbases = bvec * 16

        def fast(sg=sg, bases=bases, cbases=cbases):
          bb = bases[0]
          cb = cbases[0]
          cnt = slab_c[pl.ds(cb, 16)]
          slab_c[pl.ds(cb, 16)] = cnt + sixteen

          def grpf(q, _):
            for u in range(2):
              col = q * 32 + u * 16
              vs = [buf[sg * 16 + l, pl.ds(col, 16)] for l in range(16)]
              ts = vs
              while len(ts) > 1:
                ts = [ts[i] + ts[i + 1] for i in range(0, len(ts), 2)]
              tm = vs
              while len(tm) > 1:
                tm = [jnp.maximum(tm[i], tm[i + 1])
                      for i in range(0, len(tm), 2)]
              sm = slab_s[pl.ds(bb + col, 16)]
              slab_s[pl.ds(bb + col, 16)] = sm + ts[0]
              m = slab_m[pl.ds(bb + col, 16)]
              slab_m[pl.ds(bb + col, 16)] = jnp.maximum(m, tm[0])
            return 0

          lax.fori_loop(0, HG // 2, grpf, 0)

        def slow(sg=sg, bases=bases, cbases=cbases):
          for l in range(16):
            bb = bases[l]
            cb = cbases[l]
            r = sg * 16 + l
            cnt = slab_c[pl.ds(cb, 16)]
            slab_c[pl.ds(cb, 16)] = cnt + one

            def grp_body(q, _, r=r, bb=bb):
              for u in range(4):
                col = q * 64 + u * 16
                v = buf[r, pl.ds(col, 16)]
                m = slab_m[pl.ds(bb + col, 16)]
                slab_m[pl.ds(bb + col, 16)] = jnp.maximum(m, v)
                sm = slab_s[pl.ds(bb + col, 16)]
                slab_s[pl.ds(bb + col, 16)] = sm + v
              return 0

            lax.fori_loop(0, HG // 4, grp_body, 0)

        uniform = bvec[0] == bvec[15]
        pl.when(uniform)(fast)
        pl.when(jnp.logical_not(uniform))(slow)

    def process(buf, cur_idx):
      for sg in range(T // 16):
        do_sg(buf, cur_idx, sg)

    npairs = jnp.where(s == NS - 1, LAST_PAIRS, NPAIRS)
    start_tile(0, buf0, idx0, sem0)

    def pair_body(i, _):
      t0 = 2 * i
      start_tile(t0 + 1, buf1, idx1, sem1)
      wait_tile(t0, buf0, idx0, sem0)
      process(buf0, idx0)

      @pl.when(i + 1 < npairs)
      def _():
        start_tile(t0 + 2, buf0, idx0, sem0)

      wait_tile(t0 + 1, buf1, idx1, sem1)
      process(buf1, idx1)
      return 0

    lax.fori_loop(0, npairs, pair_body, 0)

    @pl.when(s == NS - 1)
    def _():
      pltpu.sync_copy(x_hbm.at[pl.ds(TAIL0, 16), pl.ds(col0, DH)],
                      buf0.at[pl.ds(0, 16)])
      pltpu.sync_copy(b_hbm.at[pl.ds(TAIL0, 16)], idx0.at[pl.ds(0, 16)])
      do_sg(buf0, idx0, 0)

    w = (c * NS + s) * GP * DH
    pltpu.sync_copy(slab_s, out_sum.at[pl.ds(w, GP * DH)])
    pltpu.sync_copy(slab_m, out_max.at[pl.ds(w, GP * DH)])

    @pl.when(c == 0)
    def _():
      pltpu.sync_copy(slab_c, out_cnt.at[pl.ds(s * GP * 16, GP * 16)])

  return k(xp, bp)


def _tc_combine(sums, cnts, maxs, W, b):
  def body(sum_ref, cnt_ref, max_ref, w_ref, b_ref, out_ref):
    s0 = sum_ref[0, 0, :G, :]
    s1 = sum_ref[1, 0, :G, :]
    m0 = max_ref[0, 0, :G, :]
    m1 = max_ref[1, 0, :G, :]
    cnt = cnt_ref[0, :G, 0:1]
    for j in range(1, NS):
      s0 = s0 + sum_ref[0, j, :G, :]
      s1 = s1 + sum_ref[1, j, :G, :]
      m0 = jnp.maximum(m0, max_ref[0, j, :G, :])
      m1 = jnp.maximum(m1, max_ref[1, j, :G, :])
      cnt = cnt + cnt_ref[j, :G, 0:1]
    inv = 1.0 / jnp.maximum(cnt, 1.0)
    mean = jnp.concatenate([s0, s1], axis=1) * inv
    mx = jnp.concatenate([m0, m1], axis=1)
    wm = w_ref[:, 0:D] + w_ref[:, 2 * D:3 * D] + w_ref[:, 4 * D:5 * D]
    wx = w_ref[:, D:2 * D] + w_ref[:, 3 * D:4 * D] + w_ref[:, 5 * D:6 * D]
    dn = (((1,), (1,)), ((), ()))
    out = lax.dot_general(mean, wm, dn, preferred_element_type=jnp.float32)
    out += lax.dot_general(mx, wx, dn, preferred_element_type=jnp.float32)
    out_ref[...] = out + b_ref[0, :]

  return pl.pallas_call(
      body,
      out_shape=jax.ShapeDtypeStruct((G, D), jnp.float32),
  )(sums, cnts, maxs, W, b.reshape(1, D))


def kernel(x, batch, W, b):
  sums, maxs, cnts = _sc_pool(x, batch)
  return _tc_combine(sums.reshape(NC, NS, GP, DH),
                     cnts.reshape(NS, GP, 16),
                     maxs.reshape(NC, NS, GP, DH), W, b)


# speedup vs baseline: 10.9531x; 1.3553x over previous
import functools

import jax
import jax.numpy as jnp
from jax import lax
from jax.experimental import pallas as pl
from jax.experimental.pallas import tpu as pltpu
from jax.experimental.pallas import tpu_sc as plsc

N = 50000
D = 512
G = 128

NC = 2
NS = 16

DH = D // NC
T = 64
RPW = 3136
NTILES = RPW // T
NPAIRS = 24
LAST_PAIRS = 23
TAIL0 = 49984
GP = 128

HG = DH // 16


def _sc_pool(xp, bp):
  mesh = plsc.VectorSubcoreMesh(core_axis_name="c", subcore_axis_name="s")

  @functools.partial(
      pl.kernel,
      out_type=(
          jax.ShapeDtypeStruct((NC * NS * GP * DH,), jnp.float32),
          jax.ShapeDtypeStruct((NC * NS * GP * DH,), jnp.float32),
          jax.ShapeDtypeStruct((NS * GP * 16,), jnp.float32),
      ),
      mesh=mesh,
      scratch_types=dict(
          buf0=pltpu.VMEM((T, DH), jnp.float32),
          buf1=pltpu.VMEM((T, DH), jnp.float32),
          idx0=pltpu.VMEM((T,), jnp.int32),
          idx1=pltpu.VMEM((T,), jnp.int32),
          slab_s=pltpu.VMEM((GP * DH,), jnp.float32),
          slab_m=pltpu.VMEM((GP * DH,), jnp.float32),
          slab_c=pltpu.VMEM((GP * 16,), jnp.float32),
          sem0=pltpu.SemaphoreType.DMA,
          sem1=pltpu.SemaphoreType.DMA,
      ),
  )
  def k(x_hbm, b_hbm, out_sum, out_max, out_cnt,
        buf0, buf1, idx0, idx1, slab_s, slab_m, slab_c, sem0, sem1):
    c = lax.axis_index("c")
    s = lax.axis_index("s")
    col0 = c * DH

    zero = jnp.zeros((16,), jnp.float32)
    neg = jnp.full((16,), -jnp.inf, jnp.float32)

    def fill(i, _):
      slab_s[pl.ds(i * 16, 16)] = zero
      slab_m[pl.ds(i * 16, 16)] = neg
      return 0

    lax.fori_loop(0, GP * HG, fill, 0)

    def fillc(i, _):
      slab_c[pl.ds(i * 16, 16)] = zero
      return 0

    lax.fori_loop(0, GP, fillc, 0)

    one = jnp.full((16,), 1.0, jnp.float32)

    def start_tile(t, buf, idx, sem):
      rowbase = s * RPW + t * T
      pltpu.async_copy(x_hbm.at[pl.ds(rowbase, T), pl.ds(col0, DH)], buf, sem)
      pltpu.async_copy(b_hbm.at[pl.ds(rowbase, T)], idx, sem)

    def wait_tile(t, buf, idx, sem):
      rowbase = s * RPW + t * T
      pltpu.make_async_copy(
          x_hbm.at[pl.ds(rowbase, T), pl.ds(col0, DH)], buf, sem).wait()
      pltpu.make_async_copy(b_hbm.at[pl.ds(rowbase, T)], idx, sem).wait()

    sixteen = jnp.full((16,), 16.0, jnp.float32)

    def do_sg(buf, cur_idx, sg):
      if True:
        bvec = cur_idx[pl.ds(sg * 16, 16)]
        bases = bvec * DH
        cbases = bvec * 16

        def fast(sg=sg, bases=bases, cbases=cbases):
          bb = bases[0]
          cb = cbases[0]
          cnt = slab_c[pl.ds(cb, 16)]
          slab_c[pl.ds(cb, 16)] = cnt + sixteen

          def grpf(q, _):
            for u in range(2):
              col = q * 32 + u * 16
              vs = [buf[sg * 16 + l, pl.ds(col, 16)] for l in range(16)]
              ts = vs
              while len(ts) > 1:
                ts = [ts[i] + ts[i + 1] for i in range(0, len(ts), 2)]
              tm = vs
              while len(tm) > 1:
                tm = [jnp.maximum(tm[i], tm[i + 1])
                      for i in range(0, len(tm), 2)]
              sm = slab_s[pl.ds(bb + col, 16)]
              slab_s[pl.ds(bb + col, 16)] = sm + ts[0]
              m = slab_m[pl.ds(bb + col, 16)]
              slab_m[pl.ds(bb + col, 16)] = jnp.maximum(m, tm[0])
            return 0

          lax.fori_loop(0, HG // 2, grpf, 0)

        def slow(sg=sg, bases=bases, cbases=cbases):
          for l in range(16):
            bb = bases[l]
            cb = cbases[l]
            r = sg * 16 + l
            cnt = slab_c[pl.ds(cb, 16)]
            slab_c[pl.ds(cb, 16)] = cnt + one

            def grp_body(q, _, r=r, bb=bb):
              for u in range(4):
                col = q * 64 + u * 16
                v = buf[r, pl.ds(col, 16)]
                m = slab_m[pl.ds(bb + col, 16)]
                slab_m[pl.ds(bb + col, 16)] = jnp.maximum(m, v)
                sm = slab_s[pl.ds(bb + col, 16)]
                slab_s[pl.ds(bb + col, 16)] = sm + v
              return 0

            lax.fori_loop(0, HG // 4, grp_body, 0)

        uniform = bvec[0] == bvec[15]
        pl.when(uniform)(fast)
        pl.when(jnp.logical_not(uniform))(slow)

    def process(buf, cur_idx):
      def sg_body(sg, _):
        do_sg(buf, cur_idx, sg)
        return 0

      lax.fori_loop(0, T // 16, sg_body, 0)

    last = s == NS - 1
    npairs = jnp.where(last, LAST_PAIRS, NPAIRS)
    nstart = jnp.where(last, LAST_PAIRS, NPAIRS + 1)
    start_tile(0, buf0, idx0, sem0)

    def pair_body(i, _):
      t0 = 2 * i
      start_tile(t0 + 1, buf1, idx1, sem1)
      wait_tile(t0, buf0, idx0, sem0)
      process(buf0, idx0)

      @pl.when(i + 1 < nstart)
      def _():
        start_tile(t0 + 2, buf0, idx0, sem0)

      wait_tile(t0 + 1, buf1, idx1, sem1)
      process(buf1, idx1)
      return 0

    lax.fori_loop(0, npairs, pair_body, 0)

    @pl.when(jnp.logical_not(last))
    def _():
      wait_tile(NTILES - 1, buf0, idx0, sem0)
      process(buf0, idx0)

    @pl.when(last)
    def _():
      pltpu.sync_copy(x_hbm.at[pl.ds(TAIL0, 16), pl.ds(col0, DH)],
                      buf0.at[pl.ds(0, 16)])
      pltpu.sync_copy(b_hbm.at[pl.ds(TAIL0, 16)], idx0.at[pl.ds(0, 16)])
      do_sg(buf0, idx0, 0)

    w = (c * NS + s) * GP * DH
    pltpu.sync_copy(slab_s, out_sum.at[pl.ds(w, GP * DH)])
    pltpu.sync_copy(slab_m, out_max.at[pl.ds(w, GP * DH)])

    @pl.when(c == 0)
    def _():
      pltpu.sync_copy(slab_c, out_cnt.at[pl.ds(s * GP * 16, GP * 16)])

  return k(xp, bp)


def _tc_combine(sums, cnts, maxs, W, b):
  def body(sum_ref, cnt_ref, max_ref, w_ref, b_ref, out_ref):
    s0 = sum_ref[0, 0, :G, :]
    s1 = sum_ref[1, 0, :G, :]
    m0 = max_ref[0, 0, :G, :]
    m1 = max_ref[1, 0, :G, :]
    cnt = cnt_ref[0, :G, 0:1]
    for j in range(1, NS):
      s0 = s0 + sum_ref[0, j, :G, :]
      s1 = s1 + sum_ref[1, j, :G, :]
      m0 = jnp.maximum(m0, max_ref[0, j, :G, :])
      m1 = jnp.maximum(m1, max_ref[1, j, :G, :])
      cnt = cnt + cnt_ref[j, :G, 0:1]
    inv = 1.0 / jnp.maximum(cnt, 1.0)
    mean = jnp.concatenate([s0, s1], axis=1) * inv
    mx = jnp.concatenate([m0, m1], axis=1)
    wm = w_ref[:, 0:D] + w_ref[:, 2 * D:3 * D] + w_ref[:, 4 * D:5 * D]
    wx = w_ref[:, D:2 * D] + w_ref[:, 3 * D:4 * D] + w_ref[:, 5 * D:6 * D]
    dn = (((1,), (1,)), ((), ()))
    out = lax.dot_general(mean, wm, dn, preferred_element_type=jnp.float32)
    out += lax.dot_general(mx, wx, dn, preferred_element_type=jnp.float32)
    out_ref[...] = out + b_ref[0, :]

  return pl.pallas_call(
      body,
      out_shape=jax.ShapeDtypeStruct((G, D), jnp.float32),
  )(sums, cnts, maxs, W, b.reshape(1, D))


def kernel(x, batch, W, b):
  sums, maxs, cnts = _sc_pool(x, batch)
  return _tc_combine(sums.reshape(NC, NS, GP, DH),
                     cnts.reshape(NS, GP, 16),
                     maxs.reshape(NC, NS, GP, DH), W, b)


# speedup vs baseline: 11.3870x; 1.0396x over previous
import functools

import jax
import jax.numpy as jnp
from jax import lax
from jax.experimental import pallas as pl
from jax.experimental.pallas import tpu as pltpu
from jax.experimental.pallas import tpu_sc as plsc

N = 50000
D = 512
G = 128

NC = 2
NS = 16

DH = D // NC
T = 64
RPW = 3136
NTILES = RPW // T
NPAIRS = 24
LAST_PAIRS = 23
TAIL0 = 49984
GP = 128

HG = DH // 16


def _sc_pool(xp, bp):
  mesh = plsc.VectorSubcoreMesh(core_axis_name="c", subcore_axis_name="s")

  @functools.partial(
      pl.kernel,
      out_type=(
          jax.ShapeDtypeStruct((NC * NS * GP * DH,), jnp.float32),
          jax.ShapeDtypeStruct((NC * NS * GP * DH,), jnp.float32),
          jax.ShapeDtypeStruct((NS * GP * 16,), jnp.float32),
      ),
      mesh=mesh,
      scratch_types=dict(
          buf0=pltpu.VMEM((T, DH), jnp.float32),
          buf1=pltpu.VMEM((T, DH), jnp.float32),
          idx0=pltpu.VMEM((T,), jnp.int32),
          idx1=pltpu.VMEM((T,), jnp.int32),
          slab_s=pltpu.VMEM((GP * DH,), jnp.float32),
          slab_m=pltpu.VMEM((GP * DH,), jnp.float32),
          slab_c=pltpu.VMEM((GP * 16,), jnp.float32),
          sem0=pltpu.SemaphoreType.DMA,
          sem1=pltpu.SemaphoreType.DMA,
      ),
  )
  def k(x_hbm, b_hbm, out_sum, out_max, out_cnt,
        buf0, buf1, idx0, idx1, slab_s, slab_m, slab_c, sem0, sem1):
    c = lax.axis_index("c")
    s = lax.axis_index("s")
    col0 = c * DH

    zero = jnp.zeros((16,), jnp.float32)
    neg = jnp.full((16,), -jnp.inf, jnp.float32)

    def fill(i, _):
      for u in range(8):
        slab_s[pl.ds(i * 128 + u * 16, 16)] = zero
        slab_m[pl.ds(i * 128 + u * 16, 16)] = neg
      return 0

    lax.fori_loop(0, GP * HG // 8, fill, 0)

    def fillc(i, _):
      for u in range(8):
        slab_c[pl.ds(i * 128 + u * 16, 16)] = zero
      return 0

    lax.fori_loop(0, GP // 8, fillc, 0)

    one = jnp.full((16,), 1.0, jnp.float32)

    def start_tile(t, buf, idx, sem):
      rowbase = s * RPW + t * T
      pltpu.async_copy(x_hbm.at[pl.ds(rowbase, T), pl.ds(col0, DH)], buf, sem)
      pltpu.async_copy(b_hbm.at[pl.ds(rowbase, T)], idx, sem)

    def wait_tile(t, buf, idx, sem):
      rowbase = s * RPW + t * T
      pltpu.make_async_copy(
          x_hbm.at[pl.ds(rowbase, T), pl.ds(col0, DH)], buf, sem).wait()
      pltpu.make_async_copy(b_hbm.at[pl.ds(rowbase, T)], idx, sem).wait()

    sixteen = jnp.full((16,), 16.0, jnp.float32)

    def do_sg(buf, cur_idx, sg):
      if True:
        bvec = cur_idx[pl.ds(sg * 16, 16)]
        bases = bvec * DH
        cbases = bvec * 16

        def fast(sg=sg, bases=bases, cbases=cbases):
          bb = bases[0]
          cb = cbases[0]
          cnt = slab_c[pl.ds(cb, 16)]
          slab_c[pl.ds(cb, 16)] = cnt + sixteen

          def grpf(q, _):
            for u in range(2):
              col = q * 32 + u * 16
              vs = [buf[sg * 16 + l, pl.ds(col, 16)] for l in range(16)]
              ts = vs
              while len(ts) > 1:
                ts = [ts[i] + ts[i + 1] for i in range(0, len(ts), 2)]
              tm = vs
              while len(tm) > 1:
                tm = [jnp.maximum(tm[i], tm[i + 1])
                      for i in range(0, len(tm), 2)]
              sm = slab_s[pl.ds(bb + col, 16)]
              slab_s[pl.ds(bb + col, 16)] = sm + ts[0]
              m = slab_m[pl.ds(bb + col, 16)]
              slab_m[pl.ds(bb + col, 16)] = jnp.maximum(m, tm[0])
            return 0

          lax.fori_loop(0, HG // 2, grpf, 0)

        def slow(sg=sg, bases=bases, cbases=cbases):
          for l in range(16):
            bb = bases[l]
            cb = cbases[l]
            r = sg * 16 + l
            cnt = slab_c[pl.ds(cb, 16)]
            slab_c[pl.ds(cb, 16)] = cnt + one

            def grp_body(q, _, r=r, bb=bb):
              for u in range(4):
                col = q * 64 + u * 16
                v = buf[r, pl.ds(col, 16)]
                m = slab_m[pl.ds(bb + col, 16)]
                slab_m[pl.ds(bb + col, 16)] = jnp.maximum(m, v)
                sm = slab_s[pl.ds(bb + col, 16)]
                slab_s[pl.ds(bb + col, 16)] = sm + v
              return 0

            lax.fori_loop(0, HG // 4, grp_body, 0)

        uniform = bvec[0] == bvec[15]
        pl.when(uniform)(fast)
        pl.when(jnp.logical_not(uniform))(slow)

    def process(buf, cur_idx):
      def sg_body(sg, _):
        do_sg(buf, cur_idx, sg)
        return 0

      lax.fori_loop(0, T // 16, sg_body, 0)

    last = s == NS - 1
    npairs = jnp.where(last, LAST_PAIRS, NPAIRS)
    nstart = jnp.where(last, LAST_PAIRS, NPAIRS + 1)
    start_tile(0, buf0, idx0, sem0)

    def pair_body(i, _):
      t0 = 2 * i
      start_tile(t0 + 1, buf1, idx1, sem1)
      wait_tile(t0, buf0, idx0, sem0)
      process(buf0, idx0)

      @pl.when(i + 1 < nstart)
      def _():
        start_tile(t0 + 2, buf0, idx0, sem0)

      wait_tile(t0 + 1, buf1, idx1, sem1)
      process(buf1, idx1)
      return 0

    lax.fori_loop(0, npairs, pair_body, 0)

    @pl.when(jnp.logical_not(last))
    def _():
      wait_tile(NTILES - 1, buf0, idx0, sem0)
      process(buf0, idx0)

    @pl.when(last)
    def _():
      pltpu.sync_copy(x_hbm.at[pl.ds(TAIL0, 16), pl.ds(col0, DH)],
                      buf0.at[pl.ds(0, 16)])
      pltpu.sync_copy(b_hbm.at[pl.ds(TAIL0, 16)], idx0.at[pl.ds(0, 16)])
      do_sg(buf0, idx0, 0)

    w = (c * NS + s) * GP * DH
    h1 = pltpu.async_copy(slab_s, out_sum.at[pl.ds(w, GP * DH)], sem0)
    h2 = pltpu.async_copy(slab_m, out_max.at[pl.ds(w, GP * DH)], sem1)

    @pl.when(c == 0)
    def _():
      pltpu.sync_copy(slab_c, out_cnt.at[pl.ds(s * GP * 16, GP * 16)])

    h1.wait()
    h2.wait()

  return k(xp, bp)


def _tc_combine(sums, cnts, maxs, W, b):
  def body(sum_ref, cnt_ref, max_ref, w_ref, b_ref, out_ref):
    s0 = sum_ref[0, 0, :G, :]
    s1 = sum_ref[1, 0, :G, :]
    m0 = max_ref[0, 0, :G, :]
    m1 = max_ref[1, 0, :G, :]
    cnt = cnt_ref[0, :G, 0:1]
    for j in range(1, NS):
      s0 = s0 + sum_ref[0, j, :G, :]
      s1 = s1 + sum_ref[1, j, :G, :]
      m0 = jnp.maximum(m0, max_ref[0, j, :G, :])
      m1 = jnp.maximum(m1, max_ref[1, j, :G, :])
      cnt = cnt + cnt_ref[j, :G, 0:1]
    inv = 1.0 / jnp.maximum(cnt, 1.0)
    mean = jnp.concatenate([s0, s1], axis=1) * inv
    mx = jnp.concatenate([m0, m1], axis=1)
    wm = w_ref[:, 0:D] + w_ref[:, 2 * D:3 * D] + w_ref[:, 4 * D:5 * D]
    wx = w_ref[:, D:2 * D] + w_ref[:, 3 * D:4 * D] + w_ref[:, 5 * D:6 * D]
    dn = (((1,), (1,)), ((), ()))
    out = lax.dot_general(mean, wm, dn, preferred_element_type=jnp.float32)
    out += lax.dot_general(mx, wx, dn, preferred_element_type=jnp.float32)
    out_ref[...] = out + b_ref[0, :]

  return pl.pallas_call(
      body,
      out_shape=jax.ShapeDtypeStruct((G, D), jnp.float32),
  )(sums, cnts, maxs, W, b.reshape(1, D))


def kernel(x, batch, W, b):
  sums, maxs, cnts = _sc_pool(x, batch)
  return _tc_combine(sums.reshape(NC, NS, GP, DH),
                     cnts.reshape(NS, GP, 16),
                     maxs.reshape(NC, NS, GP, DH), W, b)


# speedup vs baseline: 11.5353x; 1.0130x over previous
import functools

import jax
import jax.numpy as jnp
from jax import lax
from jax.experimental import pallas as pl
from jax.experimental.pallas import tpu as pltpu
from jax.experimental.pallas import tpu_sc as plsc

N = 50000
D = 512
G = 128

NC = 2
NS = 16

DH = D // NC
T = 64
RPW = 3136
NTILES = RPW // T
NPAIRS = 24
LAST_PAIRS = 23
TAIL0 = 49984
GP = 128

HG = DH // 16


def _sc_pool(xp, bp):
  mesh = plsc.VectorSubcoreMesh(core_axis_name="c", subcore_axis_name="s")

  @functools.partial(
      pl.kernel,
      out_type=(
          jax.ShapeDtypeStruct((NC * NS * GP * DH,), jnp.float32),
          jax.ShapeDtypeStruct((NC * NS * GP * DH,), jnp.float32),
          jax.ShapeDtypeStruct((NS * GP * 16,), jnp.float32),
      ),
      mesh=mesh,
      scratch_types=dict(
          buf0=pltpu.VMEM((T, DH), jnp.float32),
          buf1=pltpu.VMEM((T, DH), jnp.float32),
          idx0=pltpu.VMEM((T,), jnp.int32),
          idx1=pltpu.VMEM((T,), jnp.int32),
          slab_s=pltpu.VMEM((GP * DH,), jnp.float32),
          slab_m=pltpu.VMEM((GP * DH,), jnp.float32),
          slab_c=pltpu.VMEM((GP * 16,), jnp.float32),
          sem0=pltpu.SemaphoreType.DMA,
          sem1=pltpu.SemaphoreType.DMA,
      ),
  )
  def k(x_hbm, b_hbm, out_sum, out_max, out_cnt,
        buf0, buf1, idx0, idx1, slab_s, slab_m, slab_c, sem0, sem1):
    c = lax.axis_index("c")
    s = lax.axis_index("s")
    col0 = c * DH

    zero = jnp.zeros((16,), jnp.float32)
    neg = jnp.full((16,), -jnp.inf, jnp.float32)

    def fill(i, _):
      for u in range(8):
        slab_s[pl.ds(i * 128 + u * 16, 16)] = zero
        slab_m[pl.ds(i * 128 + u * 16, 16)] = neg
      return 0

    lax.fori_loop(0, GP * HG // 8, fill, 0)

    def fillc(i, _):
      for u in range(8):
        slab_c[pl.ds(i * 128 + u * 16, 16)] = zero
      return 0

    lax.fori_loop(0, GP // 8, fillc, 0)

    one = jnp.full((16,), 1.0, jnp.float32)

    def start_tile(t, buf, idx, sem):
      rowbase = s * RPW + t * T
      pltpu.async_copy(x_hbm.at[pl.ds(rowbase, T), pl.ds(col0, DH)], buf, sem)
      pltpu.async_copy(b_hbm.at[pl.ds(rowbase, T)], idx, sem)

    def wait_tile(t, buf, idx, sem):
      rowbase = s * RPW + t * T
      pltpu.make_async_copy(
          x_hbm.at[pl.ds(rowbase, T), pl.ds(col0, DH)], buf, sem).wait()
      pltpu.make_async_copy(b_hbm.at[pl.ds(rowbase, T)], idx, sem).wait()

    sixteen = jnp.full((16,), 16.0, jnp.float32)

    def do_range(buf, sg, bvec, bases, cbases, r0, n):
      bb = bases[r0]
      cb = cbases[r0]

      if n == 1:
        cnt = slab_c[pl.ds(cb, 16)]
        slab_c[pl.ds(cb, 16)] = cnt + one

        def grp1(q, _):
          for u in range(4):
            col = q * 64 + u * 16
            v = buf[sg * 16 + r0, pl.ds(col, 16)]
            m = slab_m[pl.ds(bb + col, 16)]
            slab_m[pl.ds(bb + col, 16)] = jnp.maximum(m, v)
            sm = slab_s[pl.ds(bb + col, 16)]
            slab_s[pl.ds(bb + col, 16)] = sm + v
          return 0

        lax.fori_loop(0, HG // 4, grp1, 0)
        return

      def fast():
        cnt = slab_c[pl.ds(cb, 16)]
        slab_c[pl.ds(cb, 16)] = cnt + jnp.full((16,), float(n), jnp.float32)
        unroll = 2 if n >= 8 else 4

        def grpf(q, _):
          for u in range(unroll):
            col = q * unroll * 16 + u * 16
            vs = [buf[sg * 16 + r0 + l, pl.ds(col, 16)] for l in range(n)]
            ts = vs
            while len(ts) > 1:
              ts = [ts[i] + ts[i + 1] for i in range(0, len(ts) - 1, 2)] + (
                  [ts[-1]] if len(ts) % 2 else [])
            tm = vs
            while len(tm) > 1:
              tm = [jnp.maximum(tm[i], tm[i + 1])
                    for i in range(0, len(tm) - 1, 2)] + (
                  [tm[-1]] if len(tm) % 2 else [])
            sm = slab_s[pl.ds(bb + col, 16)]
            slab_s[pl.ds(bb + col, 16)] = sm + ts[0]
            m = slab_m[pl.ds(bb + col, 16)]
            slab_m[pl.ds(bb + col, 16)] = jnp.maximum(m, tm[0])
          return 0

        lax.fori_loop(0, HG // unroll, grpf, 0)

      if n > 8:
        def split():
          do_range(buf, sg, bvec, bases, cbases, r0, n // 2)
          do_range(buf, sg, bvec, bases, cbases, r0 + n // 2, n - n // 2)
      else:
        def split():
          for l in range(n):
            do_range(buf, sg, bvec, bases, cbases, r0 + l, 1)

      uniform = bvec[r0] == bvec[r0 + n - 1]
      pl.when(uniform)(fast)
      pl.when(jnp.logical_not(uniform))(split)

    def do_sg(buf, cur_idx, sg):
      bvec = cur_idx[pl.ds(sg * 16, 16)]
      bases = bvec * DH
      cbases = bvec * 16
      do_range(buf, sg, bvec, bases, cbases, 0, 16)

    def process(buf, cur_idx):
      def sg_body(sg, _):
        do_sg(buf, cur_idx, sg)
        return 0

      lax.fori_loop(0, T // 16, sg_body, 0)

    last = s == NS - 1
    npairs = jnp.where(last, LAST_PAIRS, NPAIRS)
    nstart = jnp.where(last, LAST_PAIRS, NPAIRS + 1)
    start_tile(0, buf0, idx0, sem0)

    def pair_body(i, _):
      t0 = 2 * i
      start_tile(t0 + 1, buf1, idx1, sem1)
      wait_tile(t0, buf0, idx0, sem0)
      process(buf0, idx0)

      @pl.when(i + 1 < nstart)
      def _():
        start_tile(t0 + 2, buf0, idx0, sem0)

      wait_tile(t0 + 1, buf1, idx1, sem1)
      process(buf1, idx1)
      return 0

    lax.fori_loop(0, npairs, pair_body, 0)

    @pl.when(jnp.logical_not(last))
    def _():
      wait_tile(NTILES - 1, buf0, idx0, sem0)
      process(buf0, idx0)

    @pl.when(last)
    def _():
      pltpu.sync_copy(x_hbm.at[pl.ds(TAIL0, 16), pl.ds(col0, DH)],
                      buf0.at[pl.ds(0, 16)])
      pltpu.sync_copy(b_hbm.at[pl.ds(TAIL0, 16)], idx0.at[pl.ds(0, 16)])
      do_sg(buf0, idx0, 0)

    w = (c * NS + s) * GP * DH
    h1 = pltpu.async_copy(slab_s, out_sum.at[pl.ds(w, GP * DH)], sem0)
    h2 = pltpu.async_copy(slab_m, out_max.at[pl.ds(w, GP * DH)], sem1)

    @pl.when(c == 0)
    def _():
      pltpu.sync_copy(slab_c, out_cnt.at[pl.ds(s * GP * 16, GP * 16)])

    h1.wait()
    h2.wait()

  return k(xp, bp)


def _tc_combine(sums, cnts, maxs, W, b):
  def body(sum_ref, cnt_ref, max_ref, w_ref, b_ref, out_ref):
    s0 = sum_ref[0, 0, :G, :]
    s1 = sum_ref[1, 0, :G, :]
    m0 = max_ref[0, 0, :G, :]
    m1 = max_ref[1, 0, :G, :]
    cnt = cnt_ref[0, :G, 0:1]
    for j in range(1, NS):
      s0 = s0 + sum_ref[0, j, :G, :]
      s1 = s1 + sum_ref[1, j, :G, :]
      m0 = jnp.maximum(m0, max_ref[0, j, :G, :])
      m1 = jnp.maximum(m1, max_ref[1, j, :G, :])
      cnt = cnt + cnt_ref[j, :G, 0:1]
    inv = 1.0 / jnp.maximum(cnt, 1.0)
    mean = jnp.concatenate([s0, s1], axis=1) * inv
    mx = jnp.concatenate([m0, m1], axis=1)
    wm = w_ref[:, 0:D] + w_ref[:, 2 * D:3 * D] + w_ref[:, 4 * D:5 * D]
    wx = w_ref[:, D:2 * D] + w_ref[:, 3 * D:4 * D] + w_ref[:, 5 * D:6 * D]
    dn = (((1,), (1,)), ((), ()))
    out = lax.dot_general(mean, wm, dn, preferred_element_type=jnp.float32)
    out += lax.dot_general(mx, wx, dn, preferred_element_type=jnp.float32)
    out_ref[...] = out + b_ref[0, :]

  return pl.pallas_call(
      body,
      out_shape=jax.ShapeDtypeStruct((G, D), jnp.float32),
  )(sums, cnts, maxs, W, b.reshape(1, D))


def kernel(x, batch, W, b):
  sums, maxs, cnts = _sc_pool(x, batch)
  return _tc_combine(sums.reshape(NC, NS, GP, DH),
                     cnts.reshape(NS, GP, 16),
                     maxs.reshape(NC, NS, GP, DH), W, b)


# speedup vs baseline: 11.6299x; 1.0082x over previous
import functools

import jax
import jax.numpy as jnp
from jax import lax
from jax.experimental import pallas as pl
from jax.experimental.pallas import tpu as pltpu
from jax.experimental.pallas import tpu_sc as plsc

N = 50000
D = 512
G = 128

NC = 2
NS = 16

DH = D // NC
T = 112
RPW = 3136
NTILES = RPW // T
NPAIRS = 14
LAST_PAIRS = 13
TAIL0 = 49952
TAILSG = 3
GP = 128

HG = DH // 16


def _sc_pool(xp, bp):
  mesh = plsc.VectorSubcoreMesh(core_axis_name="c", subcore_axis_name="s")

  @functools.partial(
      pl.kernel,
      out_type=(
          jax.ShapeDtypeStruct((NC * NS * GP * DH,), jnp.float32),
          jax.ShapeDtypeStruct((NC * NS * GP * DH,), jnp.float32),
          jax.ShapeDtypeStruct((NS * GP * 16,), jnp.float32),
      ),
      mesh=mesh,
      scratch_types=dict(
          buf0=pltpu.VMEM((T, DH), jnp.float32),
          buf1=pltpu.VMEM((T, DH), jnp.float32),
          idx0=pltpu.VMEM((T,), jnp.int32),
          idx1=pltpu.VMEM((T,), jnp.int32),
          slab_s=pltpu.VMEM((GP * DH,), jnp.float32),
          slab_m=pltpu.VMEM((GP * DH,), jnp.float32),
          slab_c=pltpu.VMEM((GP * 16,), jnp.float32),
          sem0=pltpu.SemaphoreType.DMA,
          sem1=pltpu.SemaphoreType.DMA,
      ),
  )
  def k(x_hbm, b_hbm, out_sum, out_max, out_cnt,
        buf0, buf1, idx0, idx1, slab_s, slab_m, slab_c, sem0, sem1):
    c = lax.axis_index("c")
    s = lax.axis_index("s")
    col0 = c * DH

    zero = jnp.zeros((16,), jnp.float32)
    neg = jnp.full((16,), -jnp.inf, jnp.float32)

    def fill(i, _):
      for u in range(8):
        slab_s[pl.ds(i * 128 + u * 16, 16)] = zero
        slab_m[pl.ds(i * 128 + u * 16, 16)] = neg
      return 0

    lax.fori_loop(0, GP * HG // 8, fill, 0)

    def fillc(i, _):
      for u in range(8):
        slab_c[pl.ds(i * 128 + u * 16, 16)] = zero
      return 0

    lax.fori_loop(0, GP // 8, fillc, 0)

    one = jnp.full((16,), 1.0, jnp.float32)

    def start_tile(t, buf, idx, sem):
      rowbase = s * RPW + t * T
      pltpu.async_copy(x_hbm.at[pl.ds(rowbase, T), pl.ds(col0, DH)], buf, sem)
      pltpu.async_copy(b_hbm.at[pl.ds(rowbase, T)], idx, sem)

    def wait_tile(t, buf, idx, sem):
      rowbase = s * RPW + t * T
      pltpu.make_async_copy(
          x_hbm.at[pl.ds(rowbase, T), pl.ds(col0, DH)], buf, sem).wait()
      pltpu.make_async_copy(b_hbm.at[pl.ds(rowbase, T)], idx, sem).wait()

    sixteen = jnp.full((16,), 16.0, jnp.float32)

    def do_range(buf, sg, bvec, bases, cbases, r0, n):
      bb = bases[r0]
      cb = cbases[r0]

      if n == 1:
        cnt = slab_c[pl.ds(cb, 16)]
        slab_c[pl.ds(cb, 16)] = cnt + one

        def grp1(q, _):
          for u in range(4):
            col = q * 64 + u * 16
            v = buf[sg * 16 + r0, pl.ds(col, 16)]
            m = slab_m[pl.ds(bb + col, 16)]
            slab_m[pl.ds(bb + col, 16)] = jnp.maximum(m, v)
            sm = slab_s[pl.ds(bb + col, 16)]
            slab_s[pl.ds(bb + col, 16)] = sm + v
          return 0

        lax.fori_loop(0, HG // 4, grp1, 0)
        return

      def fast():
        cnt = slab_c[pl.ds(cb, 16)]
        slab_c[pl.ds(cb, 16)] = cnt + jnp.full((16,), float(n), jnp.float32)
        unroll = 2 if n >= 8 else 4

        def grpf(q, _):
          for u in range(unroll):
            col = q * unroll * 16 + u * 16
            vs = [buf[sg * 16 + r0 + l, pl.ds(col, 16)] for l in range(n)]
            ts = vs
            while len(ts) > 1:
              ts = [ts[i] + ts[i + 1] for i in range(0, len(ts) - 1, 2)] + (
                  [ts[-1]] if len(ts) % 2 else [])
            tm = vs
            while len(tm) > 1:
              tm = [jnp.maximum(tm[i], tm[i + 1])
                    for i in range(0, len(tm) - 1, 2)] + (
                  [tm[-1]] if len(tm) % 2 else [])
            sm = slab_s[pl.ds(bb + col, 16)]
            slab_s[pl.ds(bb + col, 16)] = sm + ts[0]
            m = slab_m[pl.ds(bb + col, 16)]
            slab_m[pl.ds(bb + col, 16)] = jnp.maximum(m, tm[0])
          return 0

        lax.fori_loop(0, HG // unroll, grpf, 0)

      if n > 8:
        def split():
          do_range(buf, sg, bvec, bases, cbases, r0, n // 2)
          do_range(buf, sg, bvec, bases, cbases, r0 + n // 2, n - n // 2)
      else:
        def split():
          for l in range(n):
            do_range(buf, sg, bvec, bases, cbases, r0 + l, 1)

      uniform = bvec[r0] == bvec[r0 + n - 1]
      pl.when(uniform)(fast)
      pl.when(jnp.logical_not(uniform))(split)

    def do_sg(buf, cur_idx, sg):
      bvec = cur_idx[pl.ds(sg * 16, 16)]
      bases = bvec * DH
      cbases = bvec * 16
      do_range(buf, sg, bvec, bases, cbases, 0, 16)

    def process(buf, cur_idx):
      def sg_body(sg, _):
        do_sg(buf, cur_idx, sg)
        return 0

      lax.fori_loop(0, T // 16, sg_body, 0)

    last = s == NS - 1
    npairs = jnp.where(last, LAST_PAIRS, NPAIRS)
    start_tile(0, buf0, idx0, sem0)

    def pair_body(i, _):
      t0 = 2 * i
      start_tile(t0 + 1, buf1, idx1, sem1)
      wait_tile(t0, buf0, idx0, sem0)
      process(buf0, idx0)

      @pl.when(i + 1 < npairs)
      def _():
        start_tile(t0 + 2, buf0, idx0, sem0)

      wait_tile(t0 + 1, buf1, idx1, sem1)
      process(buf1, idx1)
      return 0

    lax.fori_loop(0, npairs, pair_body, 0)

    @pl.when(last)
    def _():
      pltpu.sync_copy(x_hbm.at[pl.ds(TAIL0, TAILSG * 16), pl.ds(col0, DH)],
                      buf0.at[pl.ds(0, TAILSG * 16)])
      pltpu.sync_copy(b_hbm.at[pl.ds(TAIL0, TAILSG * 16)],
                      idx0.at[pl.ds(0, TAILSG * 16)])

      def tail_body(sg, _):
        do_sg(buf0, idx0, sg)
        return 0

      lax.fori_loop(0, TAILSG, tail_body, 0)

    w = (c * NS + s) * GP * DH
    h1 = pltpu.async_copy(slab_s, out_sum.at[pl.ds(w, GP * DH)], sem0)
    h2 = pltpu.async_copy(slab_m, out_max.at[pl.ds(w, GP * DH)], sem1)

    @pl.when(c == 0)
    def _():
      pltpu.sync_copy(slab_c, out_cnt.at[pl.ds(s * GP * 16, GP * 16)])

    h1.wait()
    h2.wait()

  return k(xp, bp)


def _tc_combine(sums, cnts, maxs, W, b):
  def body(sum_ref, cnt_ref, max_ref, w_ref, b_ref, out_ref):
    s0 = sum_ref[0, 0, :G, :]
    s1 = sum_ref[1, 0, :G, :]
    m0 = max_ref[0, 0, :G, :]
    m1 = max_ref[1, 0, :G, :]
    cnt = cnt_ref[0, :G, 0:1]
    for j in range(1, NS):
      s0 = s0 + sum_ref[0, j, :G, :]
      s1 = s1 + sum_ref[1, j, :G, :]
      m0 = jnp.maximum(m0, max_ref[0, j, :G, :])
      m1 = jnp.maximum(m1, max_ref[1, j, :G, :])
      cnt = cnt + cnt_ref[j, :G, 0:1]
    inv = 1.0 / jnp.maximum(cnt, 1.0)
    mean = jnp.concatenate([s0, s1], axis=1) * inv
    mx = jnp.concatenate([m0, m1], axis=1)
    wm = w_ref[:, 0:D] + w_ref[:, 2 * D:3 * D] + w_ref[:, 4 * D:5 * D]
    wx = w_ref[:, D:2 * D] + w_ref[:, 3 * D:4 * D] + w_ref[:, 5 * D:6 * D]
    dn = (((1,), (1,)), ((), ()))
    out = lax.dot_general(mean, wm, dn, preferred_element_type=jnp.float32)
    out += lax.dot_general(mx, wx, dn, preferred_element_type=jnp.float32)
    out_ref[...] = out + b_ref[0, :]

  return pl.pallas_call(
      body,
      out_shape=jax.ShapeDtypeStruct((G, D), jnp.float32),
  )(sums, cnts, maxs, W, b.reshape(1, D))


def kernel(x, batch, W, b):
  sums, maxs, cnts = _sc_pool(x, batch)
  return _tc_combine(sums.reshape(NC, NS, GP, DH),
                     cnts.reshape(NS, GP, 16),
                     maxs.reshape(NC, NS, GP, DH), W, b)


# speedup vs baseline: 14.4996x; 1.2468x over previous
import functools

import jax
import jax.numpy as jnp
from jax import lax
from jax.experimental import pallas as pl
from jax.experimental.pallas import tpu as pltpu
from jax.experimental.pallas import tpu_sc as plsc

N = 50000
D = 512
G = 128

NC = 2
NS = 16

DH = D // NC
T = 112
RPW = 3136
NTILES = RPW // T
NPAIRS = 14
LAST_PAIRS = 13
TAIL0 = 49952
TAILSG = 3
GP = 128

HG = DH // 16


def _sc_pool(xp, bp):
  mesh = plsc.VectorSubcoreMesh(core_axis_name="c", subcore_axis_name="s")

  @functools.partial(
      pl.kernel,
      out_type=(
          jax.ShapeDtypeStruct((NC * NS * GP * DH,), jnp.float32),
          jax.ShapeDtypeStruct((NC * NS * GP * DH,), jnp.float32),
          jax.ShapeDtypeStruct((NS * GP * 16,), jnp.float32),
      ),
      mesh=mesh,
      scratch_types=dict(
          buf0=pltpu.VMEM((T, DH), jnp.float32),
          buf1=pltpu.VMEM((T, DH), jnp.float32),
          idx0=pltpu.VMEM((T,), jnp.int32),
          idx1=pltpu.VMEM((T,), jnp.int32),
          slab_s=pltpu.VMEM((GP * DH,), jnp.float32),
          slab_m=pltpu.VMEM((GP * DH,), jnp.float32),
          slab_c=pltpu.VMEM((GP * 16,), jnp.float32),
          sem0=pltpu.SemaphoreType.DMA,
          sem1=pltpu.SemaphoreType.DMA,
      ),
  )
  def k(x_hbm, b_hbm, out_sum, out_max, out_cnt,
        buf0, buf1, idx0, idx1, slab_s, slab_m, slab_c, sem0, sem1):
    c = lax.axis_index("c")
    s = lax.axis_index("s")
    col0 = c * DH

    zero = jnp.zeros((16,), jnp.float32)
    neg = jnp.full((16,), -jnp.inf, jnp.float32)

    def fill(i, _):
      for u in range(8):
        slab_s[pl.ds(i * 128 + u * 16, 16)] = zero
        slab_m[pl.ds(i * 128 + u * 16, 16)] = neg
      return 0

    lax.fori_loop(0, GP * HG // 8, fill, 0)

    def fillc(i, _):
      for u in range(8):
        slab_c[pl.ds(i * 128 + u * 16, 16)] = zero
      return 0

    lax.fori_loop(0, GP // 8, fillc, 0)

    one = jnp.full((16,), 1.0, jnp.float32)

    def start_tile(t, buf, idx, sem):
      rowbase = s * RPW + t * T
      pltpu.async_copy(x_hbm.at[pl.ds(rowbase, T), pl.ds(col0, DH)], buf, sem)
      pltpu.async_copy(b_hbm.at[pl.ds(rowbase, T)], idx, sem)

    def wait_tile(t, buf, idx, sem):
      rowbase = s * RPW + t * T
      pltpu.make_async_copy(
          x_hbm.at[pl.ds(rowbase, T), pl.ds(col0, DH)], buf, sem).wait()
      pltpu.make_async_copy(b_hbm.at[pl.ds(rowbase, T)], idx, sem).wait()

    def do_range(buf, sg, bvec, bases, cbases, r0, n):
      bb = bases[r0]
      cb = cbases[r0]

      if n == 1:
        cnt = slab_c[pl.ds(cb, 16)]
        slab_c[pl.ds(cb, 16)] = cnt + one

        @plsc.parallel_loop(0, HG // 4)
        def grp1(q):
          for u in range(4):
            col = q * 64 + u * 16
            v = buf[sg * 16 + r0, pl.ds(col, 16)]
            m = slab_m[pl.ds(bb + col, 16)]
            slab_m[pl.ds(bb + col, 16)] = jnp.maximum(m, v)
            sm = slab_s[pl.ds(bb + col, 16)]
            slab_s[pl.ds(bb + col, 16)] = sm + v

        return

      def fast():
        cnt = slab_c[pl.ds(cb, 16)]
        slab_c[pl.ds(cb, 16)] = cnt + jnp.full((16,), float(n), jnp.float32)
        unroll = 2 if n >= 8 else 4

        @plsc.parallel_loop(0, HG // unroll)
        def grpf(q):
          for u in range(unroll):
            col = q * unroll * 16 + u * 16
            vs = [buf[sg * 16 + r0 + l, pl.ds(col, 16)] for l in range(n)]
            ts = vs
            while len(ts) > 1:
              ts = [ts[i] + ts[i + 1] for i in range(0, len(ts) - 1, 2)] + (
                  [ts[-1]] if len(ts) % 2 else [])
            tm = vs
            while len(tm) > 1:
              tm = [jnp.maximum(tm[i], tm[i + 1])
                    for i in range(0, len(tm) - 1, 2)] + (
                  [tm[-1]] if len(tm) % 2 else [])
            sm = slab_s[pl.ds(bb + col, 16)]
            slab_s[pl.ds(bb + col, 16)] = sm + ts[0]
            m = slab_m[pl.ds(bb + col, 16)]
            slab_m[pl.ds(bb + col, 16)] = jnp.maximum(m, tm[0])

      if n > 8:
        def split():
          do_range(buf, sg, bvec, bases, cbases, r0, n // 2)
          do_range(buf, sg, bvec, bases, cbases, r0 + n // 2, n - n // 2)
      else:
        def split():
          for l in range(n):
            do_range(buf, sg, bvec, bases, cbases, r0 + l, 1)

      uniform = bvec[r0] == bvec[r0 + n - 1]
      pl.when(uniform)(fast)
      pl.when(jnp.logical_not(uniform))(split)

    def do_sg(buf, cur_idx, sg):
      bvec = cur_idx[pl.ds(sg * 16, 16)]
      bases = bvec * DH
      cbases = bvec * 16
      do_range(buf, sg, bvec, bases, cbases, 0, 16)

    def process(buf, cur_idx):
      def sg_body(sg, _):
        do_sg(buf, cur_idx, sg)
        return 0

      lax.fori_loop(0, T // 16, sg_body, 0)

    last = s == NS - 1
    npairs = jnp.where(last, LAST_PAIRS, NPAIRS)
    start_tile(0, buf0, idx0, sem0)

    def pair_body(i, _):
      t0 = 2 * i
      start_tile(t0 + 1, buf1, idx1, sem1)
      wait_tile(t0, buf0, idx0, sem0)
      process(buf0, idx0)

      @pl.when(i + 1 < npairs)
      def _():
        start_tile(t0 + 2, buf0, idx0, sem0)

      wait_tile(t0 + 1, buf1, idx1, sem1)
      process(buf1, idx1)
      return 0

    lax.fori_loop(0, npairs, pair_body, 0)

    @pl.when(last)
    def _():
      pltpu.sync_copy(x_hbm.at[pl.ds(TAIL0, TAILSG * 16), pl.ds(col0, DH)],
                      buf0.at[pl.ds(0, TAILSG * 16)])
      pltpu.sync_copy(b_hbm.at[pl.ds(TAIL0, TAILSG * 16)],
                      idx0.at[pl.ds(0, TAILSG * 16)])

      def tail_body(sg, _):
        do_sg(buf0, idx0, sg)
        return 0

      lax.fori_loop(0, TAILSG, tail_body, 0)

    w = (c * NS + s) * GP * DH
    h1 = pltpu.async_copy(slab_s, out_sum.at[pl.ds(w, GP * DH)], sem0)
    h2 = pltpu.async_copy(slab_m, out_max.at[pl.ds(w, GP * DH)], sem1)

    @pl.when(c == 0)
    def _():
      pltpu.sync_copy(slab_c, out_cnt.at[pl.ds(s * GP * 16, GP * 16)])

    h1.wait()
    h2.wait()

  return k(xp, bp)


def _tc_combine(sums, cnts, maxs, W, b):
  def body(sum_ref, cnt_ref, max_ref, w_ref, b_ref, out_ref):
    s0 = sum_ref[0, 0, :G, :]
    s1 = sum_ref[1, 0, :G, :]
    m0 = max_ref[0, 0, :G, :]
    m1 = max_ref[1, 0, :G, :]
    cnt = cnt_ref[0, :G, 0:1]
    for j in range(1, NS):
      s0 = s0 + sum_ref[0, j, :G, :]
      s1 = s1 + sum_ref[1, j, :G, :]
      m0 = jnp.maximum(m0, max_ref[0, j, :G, :])
      m1 = jnp.maximum(m1, max_ref[1, j, :G, :])
      cnt = cnt + cnt_ref[j, :G, 0:1]
    inv = 1.0 / jnp.maximum(cnt, 1.0)
    mean = jnp.concatenate([s0, s1], axis=1) * inv
    mx = jnp.concatenate([m0, m1], axis=1)
    wm = w_ref[:, 0:D] + w_ref[:, 2 * D:3 * D] + w_ref[:, 4 * D:5 * D]
    wx = w_ref[:, D:2 * D] + w_ref[:, 3 * D:4 * D] + w_ref[:, 5 * D:6 * D]
    dn = (((1,), (1,)), ((), ()))
    out = lax.dot_general(mean, wm, dn, preferred_element_type=jnp.float32)
    out += lax.dot_general(mx, wx, dn, preferred_element_type=jnp.float32)
    out_ref[...] = out + b_ref[0, :]

  return pl.pallas_call(
      body,
      out_shape=jax.ShapeDtypeStruct((G, D), jnp.float32),
  )(sums, cnts, maxs, W, b.reshape(1, D))


def kernel(x, batch, W, b):
  sums, maxs, cnts = _sc_pool(x, batch)
  return _tc_combine(sums.reshape(NC, NS, GP, DH),
                     cnts.reshape(NS, GP, 16),
                     maxs.reshape(NC, NS, GP, DH), W, b)
